# initial kernel scaffold (unmeasured)
import jax
import jax.numpy as jnp
from jax import lax
from jax.experimental import pallas as pl
from jax.experimental.pallas import tpu as pltpu

N_DEV = 4
B, SQ, D_MODEL = 2, 512, 768
SKV_SH = 512
H_LOC = 8
DH = 64
SKV = N_DEV * SKV_SH


def kernel(x, Wq, K_ext, V_ext, Wo):
    def body(x_ref, wq_ref, k_ext_ref, v_ext_ref, wo_ref, out_ref,
             k_all, v_all, part_send, part_recv,
             local_sems, kv_send_sems, kv_recv_sems,
             part_send_sems, part_recv_sems):
        my = lax.axis_index("i")

        loc_k = pltpu.make_async_copy(
            k_ext_ref.at[:, :, pl.ds(H_LOC * my, H_LOC), :],
            k_all.at[N_DEV - 1], local_sems.at[0])
        loc_v = pltpu.make_async_copy(
            v_ext_ref.at[:, :, pl.ds(H_LOC * my, H_LOC), :],
            v_all.at[N_DEV - 1], local_sems.at[1])
        loc_k.start()
        loc_v.start()

        barrier_sem = pltpu.get_barrier_semaphore()
        for k in range(1, N_DEV):
            pl.semaphore_signal(
                barrier_sem, inc=1,
                device_id=((my + k) % N_DEV,),
                device_id_type=pl.DeviceIdType.MESH)
        pl.semaphore_wait(barrier_sem, N_DEV - 1)

        kv_rdmas = []
        for k in range(1, N_DEV):
            t = (my + k) % N_DEV
            j = N_DEV - 1 - k
            rk = pltpu.make_async_remote_copy(
                src_ref=k_ext_ref.at[:, :, pl.ds(H_LOC * t, H_LOC), :],
                dst_ref=k_all.at[j],
                send_sem=kv_send_sems.at[0, k - 1],
                recv_sem=kv_recv_sems.at[0, j],
                device_id=(t,), device_id_type=pl.DeviceIdType.MESH)
            rv = pltpu.make_async_remote_copy(
                src_ref=v_ext_ref.at[:, :, pl.ds(H_LOC * t, H_LOC), :],
                dst_ref=v_all.at[j],
                send_sem=kv_send_sems.at[1, k - 1],
                recv_sem=kv_recv_sems.at[1, j],
                device_id=(t,), device_id_type=pl.DeviceIdType.MESH)
            rk.start()
            rv.start()
            kv_rdmas.append((rk, rv))

        qb = lax.broadcasted_iota(jnp.int32, (SQ, SKV), 0) // 64
        col = lax.broadcasted_iota(jnp.int32, (SQ, SKV), 1)
        slot = col // SKV_SH
        kv_idx = ((my + 1 + slot) % N_DEV) * SKV_SH + col % SKV_SH
        kb = kv_idx // 64
        mask = (qb == kb) | (kb == 0) | ((qb + kb) % 3 == 0)
        bias = jnp.where(mask, 0.0, -1e9).astype(jnp.float32)

        wq = wq_ref[...].astype(jnp.bfloat16)
        q3 = []
        for b in range(B):
            qm = jnp.dot(x_ref[b].astype(jnp.bfloat16), wq,
                         preferred_element_type=jnp.float32)
            q3.append(qm.astype(jnp.bfloat16).reshape(SQ, H_LOC, DH))

        loc_k.wait()
        loc_v.wait()
        for rk, rv in kv_rdmas:
            rk.wait_recv()
            rv.wait_recv()

        for b in range(B):
            acc = jnp.zeros((SQ, D_MODEL), jnp.float32)
            for h in range(H_LOC):
                q_h = q3[b][:, h, :]
                k_full = k_all[:, b, :, h, :].reshape(SKV, DH)
                s = lax.dot_general(
                    q_h, k_full.astype(jnp.bfloat16),
                    (((1,), (1,)), ((), ())),
                    preferred_element_type=jnp.float32)
                s = s * 0.125 + bias
                m = jnp.max(s, axis=-1, keepdims=True)
                w = jnp.exp(s - m)
                denom = jnp.sum(w, axis=-1, keepdims=True)
                wn = (w / denom).astype(jnp.bfloat16)
                v_full = v_all[:, b, :, h, :].reshape(SKV, DH)
                ctx = jnp.dot(wn, v_full.astype(jnp.bfloat16),
                              preferred_element_type=jnp.float32)
                wo_h = wo_ref[h * DH:(h + 1) * DH, :].astype(jnp.bfloat16)
                acc = acc + jnp.dot(ctx.astype(jnp.bfloat16), wo_h,
                                    preferred_element_type=jnp.float32)
            out_ref[b] = acc
            part_send[b] = acc.astype(jnp.bfloat16)

        part_rdmas = []
        for k in range(1, N_DEV):
            t = (my + k) % N_DEV
            j = N_DEV - 1 - k
            r = pltpu.make_async_remote_copy(
                src_ref=part_send,
                dst_ref=part_recv.at[j],
                send_sem=part_send_sems.at[k - 1],
                recv_sem=part_recv_sems.at[j],
                device_id=(t,), device_id_type=pl.DeviceIdType.MESH)
            r.start()
            part_rdmas.append(r)
        for r in part_rdmas:
            r.wait_recv()
        tot = (part_recv[0].astype(jnp.float32)
               + part_recv[1].astype(jnp.float32)
               + part_recv[2].astype(jnp.float32))
        out_ref[...] = out_ref[...] + tot

        for rk, rv in kv_rdmas:
            rk.wait_send()
            rv.wait_send()
        for r in part_rdmas:
            r.wait_send()

    return pl.pallas_call(
        body,
        out_shape=jax.ShapeDtypeStruct((B, SQ, D_MODEL), jnp.float32),
        in_specs=[
            pl.BlockSpec(memory_space=pltpu.VMEM),
            pl.BlockSpec(memory_space=pltpu.VMEM),
            pl.BlockSpec(memory_space=pltpu.ANY),
            pl.BlockSpec(memory_space=pltpu.ANY),
            pl.BlockSpec(memory_space=pltpu.VMEM),
        ],
        out_specs=pl.BlockSpec(memory_space=pltpu.VMEM),
        scratch_shapes=[
            pltpu.VMEM((N_DEV, B, SKV_SH, H_LOC, DH), jnp.float32),
            pltpu.VMEM((N_DEV, B, SKV_SH, H_LOC, DH), jnp.float32),
            pltpu.VMEM((B, SQ, D_MODEL), jnp.bfloat16),
            pltpu.VMEM((N_DEV - 1, B, SQ, D_MODEL), jnp.bfloat16),
            pltpu.SemaphoreType.DMA((2,)),
            pltpu.SemaphoreType.DMA((2, N_DEV - 1)),
            pltpu.SemaphoreType.DMA((2, N_DEV - 1)),
            pltpu.SemaphoreType.DMA((N_DEV - 1,)),
            pltpu.SemaphoreType.DMA((N_DEV - 1,)),
        ],
        compiler_params=pltpu.CompilerParams(collective_id=0),
    )(x, Wq, K_ext, V_ext, Wo)


# baseline (device time: 196889 ns/iter reference)
import jax
import jax.numpy as jnp
from jax import lax
from jax.experimental import pallas as pl
from jax.experimental.pallas import tpu as pltpu

N_DEV = 4
B, SQ, D_MODEL = 2, 512, 768
SKV_SH = 512
H_LOC = 8
DH = 64
HD = H_LOC * DH
SKV = N_DEV * SKV_SH


def kernel(x, Wq, K_ext, V_ext, Wo):
    K2 = K_ext.reshape(B, SKV_SH, N_DEV * HD)
    V2 = V_ext.reshape(B, SKV_SH, N_DEV * HD)

    def body(x_ref, wq_ref, k_ref, v_ref, wo_ref, out_ref,
             k_all, v_all, part_send, part_recv,
             local_sems, kv_send_sems, kv_recv_sems,
             part_send_sems, part_recv_sems):
        my = lax.axis_index("i")

        loc_k = pltpu.make_async_copy(
            k_ref.at[:, :, pl.ds(HD * my, HD)],
            k_all.at[N_DEV - 1], local_sems.at[0])
        loc_v = pltpu.make_async_copy(
            v_ref.at[:, :, pl.ds(HD * my, HD)],
            v_all.at[N_DEV - 1], local_sems.at[1])
        loc_k.start()
        loc_v.start()

        barrier_sem = pltpu.get_barrier_semaphore()
        for k in range(1, N_DEV):
            pl.semaphore_signal(
                barrier_sem, inc=1,
                device_id=((my + k) % N_DEV,),
                device_id_type=pl.DeviceIdType.MESH)
        pl.semaphore_wait(barrier_sem, N_DEV - 1)

        kv_rdmas = []
        for k in range(1, N_DEV):
            t = (my + k) % N_DEV
            j = N_DEV - 1 - k
            rk = pltpu.make_async_remote_copy(
                src_ref=k_ref.at[:, :, pl.ds(HD * t, HD)],
                dst_ref=k_all.at[j],
                send_sem=kv_send_sems.at[0, k - 1],
                recv_sem=kv_recv_sems.at[0, j],
                device_id=(t,), device_id_type=pl.DeviceIdType.MESH)
            rv = pltpu.make_async_remote_copy(
                src_ref=v_ref.at[:, :, pl.ds(HD * t, HD)],
                dst_ref=v_all.at[j],
                send_sem=kv_send_sems.at[1, k - 1],
                recv_sem=kv_recv_sems.at[1, j],
                device_id=(t,), device_id_type=pl.DeviceIdType.MESH)
            rk.start()
            rv.start()
            kv_rdmas.append((rk, rv))

        qb = lax.broadcasted_iota(jnp.int32, (SQ, SKV), 0) // 64
        col = lax.broadcasted_iota(jnp.int32, (SQ, SKV), 1)
        slot = col // SKV_SH
        kv_idx = ((my + 1 + slot) % N_DEV) * SKV_SH + col % SKV_SH
        kb = kv_idx // 64
        mask = (qb == kb) | (kb == 0) | ((qb + kb) % 3 == 0)
        bias = jnp.where(mask, 0.0, -1e9).astype(jnp.float32)

        wq = wq_ref[...].astype(jnp.bfloat16)
        q3 = []
        for b in range(B):
            qm = jnp.dot(x_ref[b].astype(jnp.bfloat16), wq,
                         preferred_element_type=jnp.float32)
            q3.append(qm.astype(jnp.bfloat16).reshape(SQ, H_LOC, DH))

        loc_k.wait()
        loc_v.wait()
        for rk, rv in kv_rdmas:
            rk.wait_recv()
            rv.wait_recv()

        for b in range(B):
            acc = jnp.zeros((SQ, D_MODEL), jnp.float32)
            for h in range(H_LOC):
                q_h = q3[b][:, h, :]
                k_full = k_all[:, b, :, h * DH:(h + 1) * DH].reshape(SKV, DH)
                s = lax.dot_general(
                    q_h, k_full.astype(jnp.bfloat16),
                    (((1,), (1,)), ((), ())),
                    preferred_element_type=jnp.float32)
                s = s * 0.125 + bias
                m = jnp.max(s, axis=-1, keepdims=True)
                w = jnp.exp(s - m)
                denom = jnp.sum(w, axis=-1, keepdims=True)
                wn = (w / denom).astype(jnp.bfloat16)
                v_full = v_all[:, b, :, h * DH:(h + 1) * DH].reshape(SKV, DH)
                ctx = jnp.dot(wn, v_full.astype(jnp.bfloat16),
                              preferred_element_type=jnp.float32)
                wo_h = wo_ref[h * DH:(h + 1) * DH, :].astype(jnp.bfloat16)
                acc = acc + jnp.dot(ctx.astype(jnp.bfloat16), wo_h,
                                    preferred_element_type=jnp.float32)
            out_ref[b] = acc
            part_send[b] = acc.astype(jnp.bfloat16)

        part_rdmas = []
        for k in range(1, N_DEV):
            t = (my + k) % N_DEV
            j = N_DEV - 1 - k
            r = pltpu.make_async_remote_copy(
                src_ref=part_send,
                dst_ref=part_recv.at[j],
                send_sem=part_send_sems.at[k - 1],
                recv_sem=part_recv_sems.at[j],
                device_id=(t,), device_id_type=pl.DeviceIdType.MESH)
            r.start()
            part_rdmas.append(r)
        for r in part_rdmas:
            r.wait_recv()
        tot = (part_recv[0].astype(jnp.float32)
               + part_recv[1].astype(jnp.float32)
               + part_recv[2].astype(jnp.float32))
        out_ref[...] = out_ref[...] + tot

        for rk, rv in kv_rdmas:
            rk.wait_send()
            rv.wait_send()
        for r in part_rdmas:
            r.wait_send()

    return pl.pallas_call(
        body,
        out_shape=jax.ShapeDtypeStruct((B, SQ, D_MODEL), jnp.float32),
        in_specs=[
            pl.BlockSpec(memory_space=pltpu.MemorySpace.VMEM),
            pl.BlockSpec(memory_space=pltpu.MemorySpace.VMEM),
            pl.BlockSpec(memory_space=pltpu.MemorySpace.HBM),
            pl.BlockSpec(memory_space=pltpu.MemorySpace.HBM),
            pl.BlockSpec(memory_space=pltpu.MemorySpace.VMEM),
        ],
        out_specs=pl.BlockSpec(memory_space=pltpu.MemorySpace.VMEM),
        scratch_shapes=[
            pltpu.MemorySpace.VMEM((N_DEV, B, SKV_SH, HD), jnp.float32),
            pltpu.MemorySpace.VMEM((N_DEV, B, SKV_SH, HD), jnp.float32),
            pltpu.MemorySpace.VMEM((B, SQ, D_MODEL), jnp.bfloat16),
            pltpu.MemorySpace.VMEM((N_DEV - 1, B, SQ, D_MODEL), jnp.bfloat16),
            pltpu.SemaphoreType.DMA((2,)),
            pltpu.SemaphoreType.DMA((2, N_DEV - 1)),
            pltpu.SemaphoreType.DMA((2, N_DEV - 1)),
            pltpu.SemaphoreType.DMA((N_DEV - 1,)),
            pltpu.SemaphoreType.DMA((N_DEV - 1,)),
        ],
        compiler_params=pltpu.CompilerParams(
            collective_id=0,
            vmem_limit_bytes=100 * 1024 * 1024,
        ),
    )(x, Wq, K2, V2, Wo)


# device time: 133980 ns/iter; 1.4695x vs baseline; 1.4695x over previous
import jax
import jax.numpy as jnp
from jax import lax
from jax.experimental import pallas as pl
from jax.experimental.pallas import tpu as pltpu

N_DEV = 4
B, SQ, D_MODEL = 2, 512, 768
SKV_SH = 512
H_LOC = 8
DH = 64
HD = H_LOC * DH
SKV = N_DEV * SKV_SH


def kernel(x, Wq, K_ext, V_ext, Wo):
    K2 = K_ext.astype(jnp.bfloat16).reshape(B, SKV_SH, N_DEV * HD)
    V2 = V_ext.astype(jnp.bfloat16).reshape(B, SKV_SH, N_DEV * HD)

    def body(x_ref, wq_ref, k_ref, v_ref, wo_ref, out_ref,
             k_all, v_all, part_send, part_recv,
             local_sems, kv_send_sems, kv_recv_sems,
             part_send_sems, part_recv_sems):
        my = lax.axis_index("i")

        loc_k = pltpu.make_async_copy(
            k_ref.at[:, :, pl.ds(HD * my, HD)],
            k_all.at[N_DEV - 1], local_sems.at[0])
        loc_v = pltpu.make_async_copy(
            v_ref.at[:, :, pl.ds(HD * my, HD)],
            v_all.at[N_DEV - 1], local_sems.at[1])
        loc_k.start()
        loc_v.start()

        barrier_sem = pltpu.get_barrier_semaphore()
        for k in range(1, N_DEV):
            pl.semaphore_signal(
                barrier_sem, inc=1,
                device_id=((my + k) % N_DEV,),
                device_id_type=pl.DeviceIdType.MESH)
        pl.semaphore_wait(barrier_sem, N_DEV - 1)

        kv_rdmas = []
        for k in range(1, N_DEV):
            t = (my + k) % N_DEV
            j = N_DEV - 1 - k
            rk = pltpu.make_async_remote_copy(
                src_ref=k_ref.at[:, :, pl.ds(HD * t, HD)],
                dst_ref=k_all.at[j],
                send_sem=kv_send_sems.at[0, k - 1],
                recv_sem=kv_recv_sems.at[0, j],
                device_id=(t,), device_id_type=pl.DeviceIdType.MESH)
            rv = pltpu.make_async_remote_copy(
                src_ref=v_ref.at[:, :, pl.ds(HD * t, HD)],
                dst_ref=v_all.at[j],
                send_sem=kv_send_sems.at[1, k - 1],
                recv_sem=kv_recv_sems.at[1, j],
                device_id=(t,), device_id_type=pl.DeviceIdType.MESH)
            rk.start()
            rv.start()
            kv_rdmas.append((rk, rv))

        qb = lax.broadcasted_iota(jnp.int32, (SQ, SKV), 0) // 64
        col = lax.broadcasted_iota(jnp.int32, (SQ, SKV), 1)
        slot = col // SKV_SH
        kv_idx = ((my + 1 + slot) % N_DEV) * SKV_SH + col % SKV_SH
        kb = kv_idx // 64
        mask = (qb == kb) | (kb == 0) | ((qb + kb) % 3 == 0)
        bias = jnp.where(mask, 0.0, -1e9).astype(jnp.float32)

        wq = wq_ref[...].astype(jnp.bfloat16)
        q3 = []
        for b in range(B):
            qm = jnp.dot(x_ref[b].astype(jnp.bfloat16), wq,
                         preferred_element_type=jnp.float32)
            q3.append(qm.astype(jnp.bfloat16).reshape(SQ, H_LOC, DH))

        loc_k.wait()
        loc_v.wait()
        for rk, rv in kv_rdmas:
            rk.wait_recv()
            rv.wait_recv()

        part_rdmas = []
        for b in range(B):
            acc = jnp.zeros((SQ, D_MODEL), jnp.float32)
            for h in range(H_LOC):
                q_h = q3[b][:, h, :]
                k_full = k_all[:, b, :, h * DH:(h + 1) * DH].reshape(SKV, DH)
                s = lax.dot_general(
                    q_h, k_full,
                    (((1,), (1,)), ((), ())),
                    preferred_element_type=jnp.float32)
                s = s * 0.125 + bias
                m = jnp.max(s, axis=-1, keepdims=True)
                w = jnp.exp(s - m)
                denom = jnp.sum(w, axis=-1, keepdims=True)
                wn = (w / denom).astype(jnp.bfloat16)
                v_full = v_all[:, b, :, h * DH:(h + 1) * DH].reshape(SKV, DH)
                ctx = jnp.dot(wn, v_full,
                              preferred_element_type=jnp.float32)
                wo_h = wo_ref[h * DH:(h + 1) * DH, :].astype(jnp.bfloat16)
                acc = acc + jnp.dot(ctx.astype(jnp.bfloat16), wo_h,
                                    preferred_element_type=jnp.float32)
            out_ref[b] = acc
            part_send[b] = acc.astype(jnp.bfloat16)
            for k in range(1, N_DEV):
                t = (my + k) % N_DEV
                j = N_DEV - 1 - k
                r = pltpu.make_async_remote_copy(
                    src_ref=part_send.at[b],
                    dst_ref=part_recv.at[j, b],
                    send_sem=part_send_sems.at[k - 1, b],
                    recv_sem=part_recv_sems.at[j, b],
                    device_id=(t,), device_id_type=pl.DeviceIdType.MESH)
                r.start()
                part_rdmas.append(r)

        for r in part_rdmas:
            r.wait_recv()
        for b in range(B):
            tot = (part_recv[0, b].astype(jnp.float32)
                   + part_recv[1, b].astype(jnp.float32)
                   + part_recv[2, b].astype(jnp.float32))
            out_ref[b] = out_ref[b] + tot

        for rk, rv in kv_rdmas:
            rk.wait_send()
            rv.wait_send()
        for r in part_rdmas:
            r.wait_send()

    return pl.pallas_call(
        body,
        out_shape=jax.ShapeDtypeStruct((B, SQ, D_MODEL), jnp.float32),
        in_specs=[
            pl.BlockSpec(memory_space=pltpu.MemorySpace.VMEM),
            pl.BlockSpec(memory_space=pltpu.MemorySpace.VMEM),
            pl.BlockSpec(memory_space=pltpu.MemorySpace.HBM),
            pl.BlockSpec(memory_space=pltpu.MemorySpace.HBM),
            pl.BlockSpec(memory_space=pltpu.MemorySpace.VMEM),
        ],
        out_specs=pl.BlockSpec(memory_space=pltpu.MemorySpace.VMEM),
        scratch_shapes=[
            pltpu.MemorySpace.VMEM((N_DEV, B, SKV_SH, HD), jnp.bfloat16),
            pltpu.MemorySpace.VMEM((N_DEV, B, SKV_SH, HD), jnp.bfloat16),
            pltpu.MemorySpace.VMEM((B, SQ, D_MODEL), jnp.bfloat16),
            pltpu.MemorySpace.VMEM((N_DEV - 1, B, SQ, D_MODEL), jnp.bfloat16),
            pltpu.SemaphoreType.DMA((2,)),
            pltpu.SemaphoreType.DMA((2, N_DEV - 1)),
            pltpu.SemaphoreType.DMA((2, N_DEV - 1)),
            pltpu.SemaphoreType.DMA((N_DEV - 1, B)),
            pltpu.SemaphoreType.DMA((N_DEV - 1, B)),
        ],
        compiler_params=pltpu.CompilerParams(
            collective_id=0,
            vmem_limit_bytes=100 * 1024 * 1024,
        ),
    )(x, Wq, K2, V2, Wo)


# device time: 126667 ns/iter; 1.5544x vs baseline; 1.0577x over previous
import jax
import jax.numpy as jnp
from jax import lax
from jax.experimental import pallas as pl
from jax.experimental.pallas import tpu as pltpu

N_DEV = 4
B, SQ, D_MODEL = 2, 512, 768
SKV_SH = 512
H_LOC = 8
DH = 64
HD = H_LOC * DH
SKV = N_DEV * SKV_SH


def kernel(x, Wq, K_ext, V_ext, Wo):
    K2 = K_ext.astype(jnp.bfloat16).reshape(B, SKV_SH, N_DEV * HD)
    V2 = V_ext.astype(jnp.bfloat16).reshape(B, SKV_SH, N_DEV * HD)

    def body(x_ref, wq_ref, k_ref, v_ref, wo_ref, out_ref,
             k_all, v_all, part_send, part_recv,
             local_sems, kv_send_sems, kv_recv_sems,
             part_send_sems, part_recv_sems):
        my = lax.axis_index("i")

        loc_k = pltpu.make_async_copy(
            k_ref.at[:, :, pl.ds(HD * my, HD)],
            k_all.at[N_DEV - 1], local_sems.at[0])
        loc_v = pltpu.make_async_copy(
            v_ref.at[:, :, pl.ds(HD * my, HD)],
            v_all.at[N_DEV - 1], local_sems.at[1])
        loc_k.start()
        loc_v.start()

        barrier_sem = pltpu.get_barrier_semaphore()
        for k in range(1, N_DEV):
            pl.semaphore_signal(
                barrier_sem, inc=1,
                device_id=((my + k) % N_DEV,),
                device_id_type=pl.DeviceIdType.MESH)
        pl.semaphore_wait(barrier_sem, N_DEV - 1)

        kv_rdmas = []
        for k in range(1, N_DEV):
            t = (my + k) % N_DEV
            j = N_DEV - 1 - k
            rk = pltpu.make_async_remote_copy(
                src_ref=k_ref.at[:, :, pl.ds(HD * t, HD)],
                dst_ref=k_all.at[j],
                send_sem=kv_send_sems.at[0, k - 1],
                recv_sem=kv_recv_sems.at[0, j],
                device_id=(t,), device_id_type=pl.DeviceIdType.MESH)
            rv = pltpu.make_async_remote_copy(
                src_ref=v_ref.at[:, :, pl.ds(HD * t, HD)],
                dst_ref=v_all.at[j],
                send_sem=kv_send_sems.at[1, k - 1],
                recv_sem=kv_recv_sems.at[1, j],
                device_id=(t,), device_id_type=pl.DeviceIdType.MESH)
            rk.start()
            rv.start()
            kv_rdmas.append((rk, rv))

        qb = lax.broadcasted_iota(jnp.int32, (SQ, SKV_SH), 0) // 64
        col = lax.broadcasted_iota(jnp.int32, (SQ, SKV_SH), 1)
        biases = []
        for j in range(N_DEV):
            s_idx = (my + 1 + j) % N_DEV
            kb = (s_idx * SKV_SH + col) // 64
            m = (qb == kb) | (kb == 0) | ((qb + kb) % 3 == 0)
            biases.append(jnp.where(m, 0.0, -1e9).astype(jnp.float32))

        wq = wq_ref[...].astype(jnp.bfloat16)
        q3 = []
        for b in range(B):
            qm = jnp.dot(x_ref[b].astype(jnp.bfloat16), wq,
                         preferred_element_type=jnp.float32)
            q3.append(qm.astype(jnp.bfloat16).reshape(SQ, H_LOC, DH))

        st_m = [jnp.full((SQ, 1), -1e30, jnp.float32)] * (B * H_LOC)
        st_l = [jnp.zeros((SQ, 1), jnp.float32)] * (B * H_LOC)
        st_c = [jnp.zeros((SQ, DH), jnp.float32)] * (B * H_LOC)

        def fold_slot(j):
            for b in range(B):
                for h in range(H_LOC):
                    i = b * H_LOC + h
                    q_h = q3[b][:, h, :]
                    k_c = k_all[j, b, :, h * DH:(h + 1) * DH]
                    s = lax.dot_general(
                        q_h, k_c, (((1,), (1,)), ((), ())),
                        preferred_element_type=jnp.float32)
                    s = s * 0.125 + biases[j]
                    m_new = jnp.maximum(st_m[i],
                                        jnp.max(s, axis=-1, keepdims=True))
                    alpha = jnp.exp(st_m[i] - m_new)
                    p = jnp.exp(s - m_new)
                    st_l[i] = st_l[i] * alpha + jnp.sum(p, axis=-1,
                                                        keepdims=True)
                    v_c = v_all[j, b, :, h * DH:(h + 1) * DH]
                    pv = jnp.dot(p.astype(jnp.bfloat16), v_c,
                                 preferred_element_type=jnp.float32)
                    st_c[i] = st_c[i] * alpha + pv
                    st_m[i] = m_new

        loc_k.wait()
        loc_v.wait()
        fold_slot(N_DEV - 1)
        for k in range(1, N_DEV):
            rk, rv = kv_rdmas[k - 1]
            rk.wait_recv()
            rv.wait_recv()
            fold_slot(N_DEV - 1 - k)

        part_rdmas = []
        for b in range(B):
            acc = jnp.zeros((SQ, D_MODEL), jnp.float32)
            for h in range(H_LOC):
                i = b * H_LOC + h
                ctx = (st_c[i] / st_l[i]).astype(jnp.bfloat16)
                wo_h = wo_ref[h * DH:(h + 1) * DH, :].astype(jnp.bfloat16)
                acc = acc + jnp.dot(ctx, wo_h,
                                    preferred_element_type=jnp.float32)
            out_ref[b] = acc
            part_send[b] = acc.astype(jnp.bfloat16)
            for k in range(1, N_DEV):
                t = (my + k) % N_DEV
                j = N_DEV - 1 - k
                r = pltpu.make_async_remote_copy(
                    src_ref=part_send.at[b],
                    dst_ref=part_recv.at[j, b],
                    send_sem=part_send_sems.at[k - 1, b],
                    recv_sem=part_recv_sems.at[j, b],
                    device_id=(t,), device_id_type=pl.DeviceIdType.MESH)
                r.start()
                part_rdmas.append(r)

        for r in part_rdmas:
            r.wait_recv()
        for b in range(B):
            tot = (part_recv[0, b].astype(jnp.float32)
                   + part_recv[1, b].astype(jnp.float32)
                   + part_recv[2, b].astype(jnp.float32))
            out_ref[b] = out_ref[b] + tot

        for rk, rv in kv_rdmas:
            rk.wait_send()
            rv.wait_send()
        for r in part_rdmas:
            r.wait_send()

    return pl.pallas_call(
        body,
        out_shape=jax.ShapeDtypeStruct((B, SQ, D_MODEL), jnp.float32),
        in_specs=[
            pl.BlockSpec(memory_space=pltpu.MemorySpace.VMEM),
            pl.BlockSpec(memory_space=pltpu.MemorySpace.VMEM),
            pl.BlockSpec(memory_space=pltpu.MemorySpace.HBM),
            pl.BlockSpec(memory_space=pltpu.MemorySpace.HBM),
            pl.BlockSpec(memory_space=pltpu.MemorySpace.VMEM),
        ],
        out_specs=pl.BlockSpec(memory_space=pltpu.MemorySpace.VMEM),
        scratch_shapes=[
            pltpu.MemorySpace.VMEM((N_DEV, B, SKV_SH, HD), jnp.bfloat16),
            pltpu.MemorySpace.VMEM((N_DEV, B, SKV_SH, HD), jnp.bfloat16),
            pltpu.MemorySpace.VMEM((B, SQ, D_MODEL), jnp.bfloat16),
            pltpu.MemorySpace.VMEM((N_DEV - 1, B, SQ, D_MODEL), jnp.bfloat16),
            pltpu.SemaphoreType.DMA((2,)),
            pltpu.SemaphoreType.DMA((2, N_DEV - 1)),
            pltpu.SemaphoreType.DMA((2, N_DEV - 1)),
            pltpu.SemaphoreType.DMA((N_DEV - 1, B)),
            pltpu.SemaphoreType.DMA((N_DEV - 1, B)),
        ],
        compiler_params=pltpu.CompilerParams(
            collective_id=0,
            vmem_limit_bytes=100 * 1024 * 1024,
        ),
    )(x, Wq, K2, V2, Wo)


# device time: 118120 ns/iter; 1.6669x vs baseline; 1.0724x over previous
import jax
import jax.numpy as jnp
from jax import lax
from jax.experimental import pallas as pl
from jax.experimental.pallas import tpu as pltpu

N_DEV = 4
B, SQ, D_MODEL = 2, 512, 768
SKV_SH = 512
H_LOC = 8
DH = 64
HD = H_LOC * DH
SKV = N_DEV * SKV_SH


def kernel(x, Wq, K_ext, V_ext, Wo):
    K2 = K_ext.astype(jnp.bfloat16).reshape(B, SKV_SH, N_DEV * HD)
    V2 = V_ext.astype(jnp.bfloat16).reshape(B, SKV_SH, N_DEV * HD)

    def body(x_ref, wq_ref, k_ref, v_ref, wo_ref, out_ref,
             k_all, v_all, part_send, part_send2, part_recv,
             local_sems, kv_send_sems, kv_recv_sems,
             rd_send_sems, rd_recv_sems):
        my = lax.axis_index("i")

        loc_k = pltpu.make_async_copy(
            k_ref.at[:, :, pl.ds(HD * my, HD)],
            k_all.at[N_DEV - 1], local_sems.at[0])
        loc_v = pltpu.make_async_copy(
            v_ref.at[:, :, pl.ds(HD * my, HD)],
            v_all.at[N_DEV - 1], local_sems.at[1])
        loc_k.start()
        loc_v.start()

        barrier_sem = pltpu.get_barrier_semaphore()
        for k in range(1, N_DEV):
            pl.semaphore_signal(
                barrier_sem, inc=1,
                device_id=((my + k) % N_DEV,),
                device_id_type=pl.DeviceIdType.MESH)
        pl.semaphore_wait(barrier_sem, N_DEV - 1)

        kv_rdmas = []
        for k in range(1, N_DEV):
            t = (my + k) % N_DEV
            j = N_DEV - 1 - k
            rk = pltpu.make_async_remote_copy(
                src_ref=k_ref.at[:, :, pl.ds(HD * t, HD)],
                dst_ref=k_all.at[j],
                send_sem=kv_send_sems.at[0, k - 1],
                recv_sem=kv_recv_sems.at[0, j],
                device_id=(t,), device_id_type=pl.DeviceIdType.MESH)
            rv = pltpu.make_async_remote_copy(
                src_ref=v_ref.at[:, :, pl.ds(HD * t, HD)],
                dst_ref=v_all.at[j],
                send_sem=kv_send_sems.at[1, k - 1],
                recv_sem=kv_recv_sems.at[1, j],
                device_id=(t,), device_id_type=pl.DeviceIdType.MESH)
            rk.start()
            rv.start()
            kv_rdmas.append((rk, rv))

        qb = lax.broadcasted_iota(jnp.int32, (SQ, SKV_SH), 0) // 64
        col = lax.broadcasted_iota(jnp.int32, (SQ, SKV_SH), 1)
        biases = []
        for j in range(N_DEV):
            s_idx = (my + 1 + j) % N_DEV
            kb = (s_idx * SKV_SH + col) // 64
            m = (qb == kb) | (kb == 0) | ((qb + kb) % 3 == 0)
            biases.append(jnp.where(m, 0.0, -1e9).astype(jnp.float32))

        wq = wq_ref[...].astype(jnp.bfloat16)
        q3 = []
        for b in range(B):
            qm = jnp.dot(x_ref[b].astype(jnp.bfloat16), wq,
                         preferred_element_type=jnp.float32)
            q3.append(qm.astype(jnp.bfloat16).reshape(SQ, H_LOC, DH))

        st_m = [jnp.full((SQ, 1), -1e30, jnp.float32)] * (B * H_LOC)
        st_l = [jnp.zeros((SQ, 1), jnp.float32)] * (B * H_LOC)
        st_c = [jnp.zeros((SQ, DH), jnp.float32)] * (B * H_LOC)

        def fold_slot(j):
            for b in range(B):
                for h in range(H_LOC):
                    i = b * H_LOC + h
                    q_h = q3[b][:, h, :]
                    k_c = k_all[j, b, :, h * DH:(h + 1) * DH]
                    s = lax.dot_general(
                        q_h, k_c, (((1,), (1,)), ((), ())),
                        preferred_element_type=jnp.float32)
                    s = s * 0.125 + biases[j]
                    m_new = jnp.maximum(st_m[i],
                                        jnp.max(s, axis=-1, keepdims=True))
                    alpha = jnp.exp(st_m[i] - m_new)
                    p = jnp.exp(s - m_new)
                    st_l[i] = st_l[i] * alpha + jnp.sum(p, axis=-1,
                                                        keepdims=True)
                    v_c = v_all[j, b, :, h * DH:(h + 1) * DH]
                    pv = jnp.dot(p.astype(jnp.bfloat16), v_c,
                                 preferred_element_type=jnp.float32)
                    st_c[i] = st_c[i] * alpha + pv
                    st_m[i] = m_new

        loc_k.wait()
        loc_v.wait()
        fold_slot(N_DEV - 1)
        for k in range(1, N_DEV - 1):
            rk, rv = kv_rdmas[k - 1]
            rk.wait_recv()
            rv.wait_recv()
            fold_slot(N_DEV - 1 - k)
        rk, rv = kv_rdmas[N_DEV - 2]
        rk.wait_recv()
        rv.wait_recv()

        RC = SQ // 2
        rem = my % 2
        p1 = my + 1 - 2 * rem
        p2 = (N_DEV - 1) - my
        s1_rdmas = []
        for b in range(B):
            for rc in range(2):
                r0 = rc * RC
                acc = jnp.zeros((RC, D_MODEL), jnp.float32)
                for h in range(H_LOC):
                    i = b * H_LOC + h
                    q_h = q3[b][r0:r0 + RC, h, :]
                    k_c = k_all[0, b, :, h * DH:(h + 1) * DH]
                    s = lax.dot_general(
                        q_h, k_c, (((1,), (1,)), ((), ())),
                        preferred_element_type=jnp.float32)
                    s = s * 0.125 + biases[0][r0:r0 + RC]
                    m_prev = st_m[i][r0:r0 + RC]
                    m_new = jnp.maximum(m_prev,
                                        jnp.max(s, axis=-1, keepdims=True))
                    alpha = jnp.exp(m_prev - m_new)
                    p = jnp.exp(s - m_new)
                    l_fin = (st_l[i][r0:r0 + RC] * alpha
                             + jnp.sum(p, axis=-1, keepdims=True))
                    v_c = v_all[0, b, :, h * DH:(h + 1) * DH]
                    pv = jnp.dot(p.astype(jnp.bfloat16), v_c,
                                 preferred_element_type=jnp.float32)
                    c_fin = st_c[i][r0:r0 + RC] * alpha + pv
                    ctx = (c_fin / l_fin).astype(jnp.bfloat16)
                    wo_h = wo_ref[h * DH:(h + 1) * DH, :].astype(jnp.bfloat16)
                    acc = acc + jnp.dot(ctx, wo_h,
                                        preferred_element_type=jnp.float32)
                out_ref[b, r0:r0 + RC] = acc
                part_send[b, r0:r0 + RC] = acc.astype(jnp.bfloat16)
                pidx = b * 2 + rc
                r = pltpu.make_async_remote_copy(
                    src_ref=part_send.at[b, pl.ds(r0, RC)],
                    dst_ref=part_recv.at[0, b, pl.ds(r0, RC)],
                    send_sem=rd_send_sems.at[0, pidx],
                    recv_sem=rd_recv_sems.at[0, pidx],
                    device_id=(p1,), device_id_type=pl.DeviceIdType.MESH)
                r.start()
                s1_rdmas.append(r)

        s2_rdmas = []
        for b in range(B):
            for rc in range(2):
                r0 = rc * RC
                pidx = b * 2 + rc
                s1_rdmas[pidx].wait_recv()
                pair = (out_ref[b, r0:r0 + RC]
                        + part_recv[0, b, r0:r0 + RC].astype(jnp.float32))
                out_ref[b, r0:r0 + RC] = pair
                part_send2[b, r0:r0 + RC] = pair.astype(jnp.bfloat16)
                r = pltpu.make_async_remote_copy(
                    src_ref=part_send2.at[b, pl.ds(r0, RC)],
                    dst_ref=part_recv.at[1, b, pl.ds(r0, RC)],
                    send_sem=rd_send_sems.at[1, pidx],
                    recv_sem=rd_recv_sems.at[1, pidx],
                    device_id=(p2,), device_id_type=pl.DeviceIdType.MESH)
                r.start()
                s2_rdmas.append(r)

        for b in range(B):
            for rc in range(2):
                r0 = rc * RC
                pidx = b * 2 + rc
                s2_rdmas[pidx].wait_recv()
                out_ref[b, r0:r0 + RC] = (
                    out_ref[b, r0:r0 + RC]
                    + part_recv[1, b, r0:r0 + RC].astype(jnp.float32))

        for rk, rv in kv_rdmas:
            rk.wait_send()
            rv.wait_send()
        for r in s1_rdmas + s2_rdmas:
            r.wait_send()

    return pl.pallas_call(
        body,
        out_shape=jax.ShapeDtypeStruct((B, SQ, D_MODEL), jnp.float32),
        in_specs=[
            pl.BlockSpec(memory_space=pltpu.MemorySpace.VMEM),
            pl.BlockSpec(memory_space=pltpu.MemorySpace.VMEM),
            pl.BlockSpec(memory_space=pltpu.MemorySpace.HBM),
            pl.BlockSpec(memory_space=pltpu.MemorySpace.HBM),
            pl.BlockSpec(memory_space=pltpu.MemorySpace.VMEM),
        ],
        out_specs=pl.BlockSpec(memory_space=pltpu.MemorySpace.VMEM),
        scratch_shapes=[
            pltpu.MemorySpace.VMEM((N_DEV, B, SKV_SH, HD), jnp.bfloat16),
            pltpu.MemorySpace.VMEM((N_DEV, B, SKV_SH, HD), jnp.bfloat16),
            pltpu.MemorySpace.VMEM((B, SQ, D_MODEL), jnp.bfloat16),
            pltpu.MemorySpace.VMEM((B, SQ, D_MODEL), jnp.bfloat16),
            pltpu.MemorySpace.VMEM((2, B, SQ, D_MODEL), jnp.bfloat16),
            pltpu.SemaphoreType.DMA((2,)),
            pltpu.SemaphoreType.DMA((2, N_DEV - 1)),
            pltpu.SemaphoreType.DMA((2, N_DEV - 1)),
            pltpu.SemaphoreType.DMA((2, 2 * B)),
            pltpu.SemaphoreType.DMA((2, 2 * B)),
        ],
        compiler_params=pltpu.CompilerParams(
            collective_id=0,
            vmem_limit_bytes=100 * 1024 * 1024,
        ),
    )(x, Wq, K2, V2, Wo)


# device time: 113556 ns/iter; 1.7338x vs baseline; 1.0402x over previous
import jax
import jax.numpy as jnp
from jax import lax
from jax.experimental import pallas as pl
from jax.experimental.pallas import tpu as pltpu

N_DEV = 4
B, SQ, D_MODEL = 2, 512, 768
SKV_SH = 512
H_LOC = 8
DH = 64
HD = H_LOC * DH
SKV = N_DEV * SKV_SH


def kernel(x, Wq, K_ext, V_ext, Wo):
    K2 = K_ext.astype(jnp.bfloat16).reshape(B, SKV_SH, N_DEV * HD)
    V2 = V_ext.astype(jnp.bfloat16).reshape(B, SKV_SH, N_DEV * HD)

    def body(x_ref, wq_ref, k_ref, v_ref, wo_ref, out_ref,
             k_all, v_all, part_send, part_send2, part_recv,
             local_sems, kv_send_sems, kv_recv_sems,
             rd_send_sems, rd_recv_sems):
        my = lax.axis_index("i")

        loc_k = pltpu.make_async_copy(
            k_ref.at[:, :, pl.ds(HD * my, HD)],
            k_all.at[N_DEV - 1], local_sems.at[0])
        loc_v = pltpu.make_async_copy(
            v_ref.at[:, :, pl.ds(HD * my, HD)],
            v_all.at[N_DEV - 1], local_sems.at[1])
        loc_k.start()
        loc_v.start()

        barrier_sem = pltpu.get_barrier_semaphore()
        for k in range(1, N_DEV):
            pl.semaphore_signal(
                barrier_sem, inc=1,
                device_id=((my + k) % N_DEV,),
                device_id_type=pl.DeviceIdType.MESH)
        pl.semaphore_wait(barrier_sem, N_DEV - 1)

        kv_rdmas = []
        for k in range(1, N_DEV):
            t = (my + k) % N_DEV
            j = N_DEV - 1 - k
            rk = pltpu.make_async_remote_copy(
                src_ref=k_ref.at[:, :, pl.ds(HD * t, HD)],
                dst_ref=k_all.at[j],
                send_sem=kv_send_sems.at[0, k - 1],
                recv_sem=kv_recv_sems.at[0, j],
                device_id=(t,), device_id_type=pl.DeviceIdType.MESH)
            rv = pltpu.make_async_remote_copy(
                src_ref=v_ref.at[:, :, pl.ds(HD * t, HD)],
                dst_ref=v_all.at[j],
                send_sem=kv_send_sems.at[1, k - 1],
                recv_sem=kv_recv_sems.at[1, j],
                device_id=(t,), device_id_type=pl.DeviceIdType.MESH)
            rk.start()
            rv.start()
            kv_rdmas.append((rk, rv))

        qb = lax.broadcasted_iota(jnp.int32, (SQ, SKV_SH), 0) // 64
        col = lax.broadcasted_iota(jnp.int32, (SQ, SKV_SH), 1)
        biases = []
        for j in range(N_DEV):
            s_idx = (my + 1 + j) % N_DEV
            kb = (s_idx * SKV_SH + col) // 64
            m = (qb == kb) | (kb == 0) | ((qb + kb) % 3 == 0)
            biases.append(jnp.where(m, 0.0, -1e9).astype(jnp.float32))

        wq = wq_ref[...].astype(jnp.bfloat16)
        q3 = []
        for b in range(B):
            qm = jnp.dot(x_ref[b].astype(jnp.bfloat16), wq,
                         preferred_element_type=jnp.float32)
            q3.append(qm.astype(jnp.bfloat16).reshape(SQ, H_LOC, DH))

        st_m = [jnp.full((SQ, 1), -1e30, jnp.float32)] * (B * H_LOC)
        st_l = [jnp.zeros((SQ, 1), jnp.float32)] * (B * H_LOC)
        st_c = [jnp.zeros((SQ, DH), jnp.float32)] * (B * H_LOC)

        def fold_slot(j):
            for b in range(B):
                for h in range(H_LOC):
                    i = b * H_LOC + h
                    q_h = q3[b][:, h, :]
                    k_c = k_all[j, b, :, h * DH:(h + 1) * DH]
                    s = lax.dot_general(
                        q_h, k_c, (((1,), (1,)), ((), ())),
                        preferred_element_type=jnp.float32)
                    s = s * 0.125 + biases[j]
                    m_new = jnp.maximum(st_m[i],
                                        jnp.max(s, axis=-1, keepdims=True))
                    alpha = jnp.exp(st_m[i] - m_new)
                    p = jnp.exp(s - m_new)
                    st_l[i] = st_l[i] * alpha + jnp.sum(p, axis=-1,
                                                        keepdims=True)
                    v_c = v_all[j, b, :, h * DH:(h + 1) * DH]
                    pv = jnp.dot(p.astype(jnp.bfloat16), v_c,
                                 preferred_element_type=jnp.float32)
                    st_c[i] = st_c[i] * alpha + pv
                    st_m[i] = m_new

        loc_k.wait()
        loc_v.wait()
        fold_slot(N_DEV - 1)
        for k in range(1, N_DEV - 1):
            rk, rv = kv_rdmas[k - 1]
            rk.wait_recv()
            rv.wait_recv()
            fold_slot(N_DEV - 1 - k)
        rk, rv = kv_rdmas[N_DEV - 2]
        rk.wait_recv()
        rv.wait_recv()

        RC = SQ // 2
        rem = my % 2
        p1 = my + 1 - 2 * rem
        p2 = (N_DEV - 1) - my

        def rd_step2(pidx):
            b, rc = divmod(pidx, 2)
            r0 = rc * RC
            s1_rdmas[pidx].wait_recv()
            pair = (out_ref[b, r0:r0 + RC]
                    + part_recv[0, b, r0:r0 + RC].astype(jnp.float32))
            out_ref[b, r0:r0 + RC] = pair
            part_send2[b, r0:r0 + RC] = pair.astype(jnp.bfloat16)
            r = pltpu.make_async_remote_copy(
                src_ref=part_send2.at[b, pl.ds(r0, RC)],
                dst_ref=part_recv.at[1, b, pl.ds(r0, RC)],
                send_sem=rd_send_sems.at[1, pidx],
                recv_sem=rd_recv_sems.at[1, pidx],
                device_id=(p2,), device_id_type=pl.DeviceIdType.MESH)
            r.start()
            s2_rdmas.append(r)

        s1_rdmas = []
        s2_rdmas = []
        for b in range(B):
            for rc in range(2):
                r0 = rc * RC
                acc = jnp.zeros((RC, D_MODEL), jnp.float32)
                for h in range(H_LOC):
                    i = b * H_LOC + h
                    q_h = q3[b][r0:r0 + RC, h, :]
                    k_c = k_all[0, b, :, h * DH:(h + 1) * DH]
                    s = lax.dot_general(
                        q_h, k_c, (((1,), (1,)), ((), ())),
                        preferred_element_type=jnp.float32)
                    s = s * 0.125 + biases[0][r0:r0 + RC]
                    m_prev = st_m[i][r0:r0 + RC]
                    m_new = jnp.maximum(m_prev,
                                        jnp.max(s, axis=-1, keepdims=True))
                    alpha = jnp.exp(m_prev - m_new)
                    p = jnp.exp(s - m_new)
                    l_fin = (st_l[i][r0:r0 + RC] * alpha
                             + jnp.sum(p, axis=-1, keepdims=True))
                    v_c = v_all[0, b, :, h * DH:(h + 1) * DH]
                    pv = jnp.dot(p.astype(jnp.bfloat16), v_c,
                                 preferred_element_type=jnp.float32)
                    c_fin = st_c[i][r0:r0 + RC] * alpha + pv
                    ctx = (c_fin / l_fin).astype(jnp.bfloat16)
                    wo_h = wo_ref[h * DH:(h + 1) * DH, :].astype(jnp.bfloat16)
                    acc = acc + jnp.dot(ctx, wo_h,
                                        preferred_element_type=jnp.float32)
                out_ref[b, r0:r0 + RC] = acc
                part_send[b, r0:r0 + RC] = acc.astype(jnp.bfloat16)
                pidx = b * 2 + rc
                r = pltpu.make_async_remote_copy(
                    src_ref=part_send.at[b, pl.ds(r0, RC)],
                    dst_ref=part_recv.at[0, b, pl.ds(r0, RC)],
                    send_sem=rd_send_sems.at[0, pidx],
                    recv_sem=rd_recv_sems.at[0, pidx],
                    device_id=(p1,), device_id_type=pl.DeviceIdType.MESH)
                r.start()
                s1_rdmas.append(r)
                if pidx >= 1:
                    rd_step2(pidx - 1)
        rd_step2(2 * B - 1)

        for b in range(B):
            for rc in range(2):
                r0 = rc * RC
                pidx = b * 2 + rc
                s2_rdmas[pidx].wait_recv()
                out_ref[b, r0:r0 + RC] = (
                    out_ref[b, r0:r0 + RC]
                    + part_recv[1, b, r0:r0 + RC].astype(jnp.float32))

        for rk, rv in kv_rdmas:
            rk.wait_send()
            rv.wait_send()
        for r in s1_rdmas + s2_rdmas:
            r.wait_send()

    return pl.pallas_call(
        body,
        out_shape=jax.ShapeDtypeStruct((B, SQ, D_MODEL), jnp.float32),
        in_specs=[
            pl.BlockSpec(memory_space=pltpu.MemorySpace.VMEM),
            pl.BlockSpec(memory_space=pltpu.MemorySpace.VMEM),
            pl.BlockSpec(memory_space=pltpu.MemorySpace.HBM),
            pl.BlockSpec(memory_space=pltpu.MemorySpace.HBM),
            pl.BlockSpec(memory_space=pltpu.MemorySpace.VMEM),
        ],
        out_specs=pl.BlockSpec(memory_space=pltpu.MemorySpace.VMEM),
        scratch_shapes=[
            pltpu.MemorySpace.VMEM((N_DEV, B, SKV_SH, HD), jnp.bfloat16),
            pltpu.MemorySpace.VMEM((N_DEV, B, SKV_SH, HD), jnp.bfloat16),
            pltpu.MemorySpace.VMEM((B, SQ, D_MODEL), jnp.bfloat16),
            pltpu.MemorySpace.VMEM((B, SQ, D_MODEL), jnp.bfloat16),
            pltpu.MemorySpace.VMEM((2, B, SQ, D_MODEL), jnp.bfloat16),
            pltpu.SemaphoreType.DMA((2,)),
            pltpu.SemaphoreType.DMA((2, N_DEV - 1)),
            pltpu.SemaphoreType.DMA((2, N_DEV - 1)),
            pltpu.SemaphoreType.DMA((2, 2 * B)),
            pltpu.SemaphoreType.DMA((2, 2 * B)),
        ],
        compiler_params=pltpu.CompilerParams(
            collective_id=0,
            vmem_limit_bytes=100 * 1024 * 1024,
        ),
    )(x, Wq, K2, V2, Wo)
